# Initial kernel scaffold; baseline (speedup 1.0000x reference)
import jax
import jax.numpy as jnp
from jax.experimental import pallas as pl


def kernel(x, importance):
    raise NotImplementedError("write your pallas kernel here")


# speedup vs baseline: 15.7741x; 15.7741x over previous
import functools

import jax
import jax.numpy as jnp
from jax.experimental import pallas as pl

DIM = 32768
K = 256
BATCH = 128
ROW_BLOCK = 16


def _select_mask_kernel(x_ref, imp_ref, o_ref):
    x = x_ref[...]
    imp = imp_ref[...]
    s = jnp.abs(x) * jnp.abs(imp)
    bits = jax.lax.bitcast_convert_type(s, jnp.int32)

    lo0 = jnp.zeros((ROW_BLOCK, 1), jnp.int32)
    hi0 = jnp.full((ROW_BLOCK, 1), 0x7F800000, jnp.int32)

    def body(_, carry):
        lo, hi = carry
        mid = lo + (hi - lo + 1) // 2
        cnt = jnp.sum((bits >= mid).astype(jnp.int32), axis=1, keepdims=True)
        ge = cnt >= K
        lo = jnp.where(ge, mid, lo)
        hi = jnp.where(ge, hi, mid - 1)
        return lo, hi

    lo, _ = jax.lax.fori_loop(0, 31, body, (lo0, hi0))
    t = lo

    gt = bits > t
    n_gt = jnp.sum(gt.astype(jnp.int32), axis=1, keepdims=True)
    r = K - n_gt
    eq = bits == t
    col = jax.lax.broadcasted_iota(jnp.int32, (1, DIM), 1)

    jlo0 = jnp.full((ROW_BLOCK, 1), -1, jnp.int32)
    jhi0 = jnp.full((ROW_BLOCK, 1), DIM - 1, jnp.int32)

    def tie_body(_, carry):
        jlo, jhi = carry
        mid = jlo + (jhi - jlo) // 2
        cnt = jnp.sum((eq & (col <= mid)).astype(jnp.int32), axis=1,
                      keepdims=True)
        ok = cnt >= r
        jhi = jnp.where(ok, mid, jhi)
        jlo = jnp.where(ok, jlo, mid + 1)
        return jlo, jhi

    _, jcut = jax.lax.fori_loop(0, 16, tie_body, (jlo0, jhi0))
    keep = gt | (eq & (col <= jcut))
    o_ref[...] = jnp.where(keep, x, 0.0)


@jax.jit
def kernel(x, importance):
    imp2d = importance.reshape(1, DIM)
    grid = (BATCH // ROW_BLOCK,)
    return pl.pallas_call(
        _select_mask_kernel,
        grid=grid,
        in_specs=[
            pl.BlockSpec((ROW_BLOCK, DIM), lambda i: (i, 0)),
            pl.BlockSpec((1, DIM), lambda i: (0, 0)),
        ],
        out_specs=pl.BlockSpec((ROW_BLOCK, DIM), lambda i: (i, 0)),
        out_shape=jax.ShapeDtypeStruct((BATCH, DIM), jnp.float32),
    )(x, imp2d)


# speedup vs baseline: 24.4792x; 1.5519x over previous
import functools

import jax
import jax.numpy as jnp
from jax.experimental import pallas as pl

DIM = 32768
K = 256
BATCH = 128
ROW_BLOCK = 16


def _select_mask_kernel(x_ref, imp_ref, o_ref):
    x = x_ref[...]
    imp = imp_ref[...]
    s = jnp.abs(x) * jnp.abs(imp)
    bits = jax.lax.bitcast_convert_type(s, jnp.int32)

    chunk_max = jnp.max(bits.reshape(ROW_BLOCK, DIM // 128, 128), axis=2)
    lo0 = jnp.min(chunk_max, axis=1, keepdims=True)
    hi0 = jnp.max(chunk_max, axis=1, keepdims=True)

    def srch_cond(carry):
        lo, hi = carry
        return jnp.any(lo < hi)

    def srch_body(carry):
        lo, hi = carry
        mid = lo + (hi - lo + 1) // 2
        cnt = jnp.sum((bits >= mid).astype(jnp.int32), axis=1, keepdims=True)
        ge = cnt >= K
        lo = jnp.where(ge, mid, lo)
        hi = jnp.where(ge, hi, mid - 1)
        return lo, hi

    t, _ = jax.lax.while_loop(srch_cond, srch_body, (lo0, hi0))

    gt = bits > t
    eq = bits == t
    n_gt = jnp.sum(gt.astype(jnp.int32), axis=1, keepdims=True)
    n_eq = jnp.sum(eq.astype(jnp.int32), axis=1, keepdims=True)
    r = K - n_gt
    col = jax.lax.broadcasted_iota(jnp.int32, (1, DIM), 1)

    def tie_search():
        def tcond(carry):
            jlo, jhi = carry
            return jnp.any(jlo < jhi)

        def tbody(carry):
            jlo, jhi = carry
            mid = jlo + (jhi - jlo) // 2
            cnt = jnp.sum((eq & (col <= mid)).astype(jnp.int32), axis=1,
                          keepdims=True)
            ok = cnt >= r
            jhi = jnp.where(ok, mid, jhi)
            jlo = jnp.where(ok, jlo, mid + 1)
            return jlo, jhi

        jlo0 = jnp.full((ROW_BLOCK, 1), -1, jnp.int32)
        jhi0 = jnp.full((ROW_BLOCK, 1), DIM - 1, jnp.int32)
        jcut, _ = jax.lax.while_loop(tcond, tbody, (jlo0, jhi0))
        return jcut

    exact = jnp.all(n_gt + n_eq == K)
    jcut = jax.lax.cond(exact,
                        lambda: jnp.full((ROW_BLOCK, 1), DIM - 1, jnp.int32),
                        tie_search)
    keep = gt | (eq & (col <= jcut))
    o_ref[...] = jnp.where(keep, x, 0.0)


@jax.jit
def kernel(x, importance):
    imp2d = importance.reshape(1, DIM)
    grid = (BATCH // ROW_BLOCK,)
    return pl.pallas_call(
        _select_mask_kernel,
        grid=grid,
        in_specs=[
            pl.BlockSpec((ROW_BLOCK, DIM), lambda i: (i, 0)),
            pl.BlockSpec((1, DIM), lambda i: (0, 0)),
        ],
        out_specs=pl.BlockSpec((ROW_BLOCK, DIM), lambda i: (i, 0)),
        out_shape=jax.ShapeDtypeStruct((BATCH, DIM), jnp.float32),
    )(x, imp2d)


# speedup vs baseline: 26.9907x; 1.1026x over previous
import jax
import jax.numpy as jnp
from jax.experimental import pallas as pl

DIM = 32768
K = 256
BATCH = 128
ROW_BLOCK = 16
NT = DIM // 128


def _row_count(pred3):
    part = jnp.sum(pred3.astype(jnp.int32), axis=1)
    return jnp.sum(part, axis=1, keepdims=True)


def _select_mask_kernel(x_ref, imp_ref, o_ref):
    x = x_ref[...]
    imp = imp_ref[...]
    s = jnp.abs(x) * jnp.abs(imp)
    bits = jax.lax.bitcast_convert_type(s, jnp.int32)
    bits3 = bits.reshape(ROW_BLOCK, NT, 128)

    gmax = jnp.max(bits.reshape(ROW_BLOCK, 128, 256), axis=1)
    lo0 = jnp.min(gmax, axis=1, keepdims=True)
    hi0 = jnp.max(gmax, axis=1, keepdims=True)

    def srch_cond(carry):
        lo, hi = carry
        return jnp.any(lo < hi)

    def srch_body(carry):
        lo, hi = carry
        mid = lo + (hi - lo + 1) // 2
        cnt = _row_count(bits3 >= mid[:, :, None])
        ge = cnt >= K
        lo = jnp.where(ge, mid, lo)
        hi = jnp.where(ge, hi, mid - 1)
        return lo, hi

    t, _ = jax.lax.while_loop(srch_cond, srch_body, (lo0, hi0))

    t3 = t[:, :, None]
    n_gt = _row_count(bits3 > t3)
    n_eq = _row_count(bits3 == t3)
    r = K - n_gt
    col = jax.lax.broadcasted_iota(jnp.int32, (1, NT, 128), 1) * 128 + \
        jax.lax.broadcasted_iota(jnp.int32, (1, NT, 128), 2)
    eq3 = bits3 == t3

    def tie_search():
        def tcond(carry):
            jlo, jhi = carry
            return jnp.any(jlo < jhi)

        def tbody(carry):
            jlo, jhi = carry
            mid = jlo + (jhi - jlo) // 2
            cnt = _row_count(eq3 & (col <= mid[:, :, None]))
            ok = cnt >= r
            jhi = jnp.where(ok, mid, jhi)
            jlo = jnp.where(ok, jlo, mid + 1)
            return jlo, jhi

        jlo0 = jnp.full((ROW_BLOCK, 1), -1, jnp.int32)
        jhi0 = jnp.full((ROW_BLOCK, 1), DIM - 1, jnp.int32)
        jcut, _ = jax.lax.while_loop(tcond, tbody, (jlo0, jhi0))
        return jcut

    exact = jnp.all(n_gt + n_eq == K)
    jcut = jax.lax.cond(exact,
                        lambda: jnp.full((ROW_BLOCK, 1), DIM - 1, jnp.int32),
                        tie_search)
    keep = (bits3 > t3) | (eq3 & (col <= jcut[:, :, None]))
    out3 = jnp.where(keep, x.reshape(ROW_BLOCK, NT, 128), 0.0)
    o_ref[...] = out3.reshape(ROW_BLOCK, DIM)


@jax.jit
def kernel(x, importance):
    imp2d = importance.reshape(1, DIM)
    grid = (BATCH // ROW_BLOCK,)
    return pl.pallas_call(
        _select_mask_kernel,
        grid=grid,
        in_specs=[
            pl.BlockSpec((ROW_BLOCK, DIM), lambda i: (i, 0)),
            pl.BlockSpec((1, DIM), lambda i: (0, 0)),
        ],
        out_specs=pl.BlockSpec((ROW_BLOCK, DIM), lambda i: (i, 0)),
        out_shape=jax.ShapeDtypeStruct((BATCH, DIM), jnp.float32),
    )(x, imp2d)
